# baseline (speedup 1.0000x reference)
import functools

import jax
import jax.numpy as jnp
from jax import lax
from jax.experimental import pallas as pl
from jax.experimental.pallas import tpu as pltpu
from jax.experimental.pallas import tpu_sc as plsc

_LANES = 16
_CHUNK = 128
_RING = 16
_UBLK = 49152
_QBLK = 128


def _detile_body(n_feat, ut_ref, it_ref, ou_ref, oi_ref):
    for q in range(_UBLK // _QBLK):
        dst = pl.ds(q * n_feat, n_feat)
        src = pl.ds(q * _QBLK, _QBLK)
        ou_ref[dst, :] = ut_ref[:, src]
        oi_ref[dst, :] = it_ref[:, src]


def _mf_kernel(b_per_w, n_feat, uids_hbm, iids_hbm, ut_hbm, it_hbm,
               w_hbm, out_hbm, idx_v, widx_u, widx_i, gu_v, gi_v, w_v, out_v,
               sem):
    wid = lax.axis_index("s") * 2 + lax.axis_index("c")
    base = wid * b_per_w

    pltpu.sync_copy(uids_hbm.at[pl.ds(base, b_per_w)], idx_v.at[0])
    pltpu.sync_copy(iids_hbm.at[pl.ds(base, b_per_w)], idx_v.at[1])
    pltpu.sync_copy(w_hbm, w_v)

    n_vec = b_per_w // _LANES
    n_chunks = (b_per_w * n_feat) // _CHUNK
    per_row = _CHUNK // _LANES

    def build_body(c, carry):
        uvec = idx_v[0, pl.ds(c * _LANES, _LANES)]
        ivec = idx_v[1, pl.ds(c * _LANES, _LANES)]
        ubase = ((uvec >> 7) << 12) + (uvec & 127)
        ibase = ((ivec >> 7) << 12) + (ivec & 127)
        for f in range(n_feat):
            pos = f * b_per_w + c * _LANES
            row = pos // _CHUNK
            off = (c % per_row) * _LANES
            widx_u[row, pl.ds(off, _LANES)] = ubase + f * _QBLK
            widx_i[row, pl.ds(off, _LANES)] = ibase + f * _QBLK
        return carry

    lax.fori_loop(0, n_vec, build_body, 0)

    def fire(j):
        pltpu.make_async_copy(
            ut_hbm.at[widx_u.at[j]],
            gu_v.at[pl.ds(j * _CHUNK, _CHUNK)], sem).start()
        pltpu.make_async_copy(
            it_hbm.at[widx_i.at[j]],
            gi_v.at[pl.ds(j * _CHUNK, _CHUNK)], sem).start()

    def drain_pair():
        pltpu.make_async_copy(
            out_hbm.at[pl.ds(0, _CHUNK)],
            gu_v.at[pl.ds(0, _CHUNK)], sem).wait()
        pltpu.make_async_copy(
            out_hbm.at[pl.ds(0, _CHUNK)],
            gi_v.at[pl.ds(0, _CHUNK)], sem).wait()

    def pipe_body(j, carry):
        fire(j)

        @pl.when(j >= _RING)
        def _():
            drain_pair()

        return carry

    lax.fori_loop(0, n_chunks, pipe_body, 0)
    for _ in range(_RING):
        drain_pair()

    lane = lax.iota(jnp.int32, _LANES)
    dnums = lax.GatherDimensionNumbers(
        offset_dims=(), collapsed_slice_dims=(0,), start_index_map=(0,))

    def permute(t, p):
        return lax.gather(t, p[:, None], dnums, slice_sizes=(1,),
                          mode=lax.GatherScatterMode.PROMISE_IN_BOUNDS)

    wregs = [w_v[pl.ds(k * _LANES, _LANES)] for k in range(n_feat // _LANES)]
    ws = [permute(wregs[f // _LANES], lane * 0 + (f % _LANES))
          for f in range(n_feat)]

    def compute_body(c, carry):
        o = c * _LANES
        acc = ws[0] * gu_v[pl.ds(o, _LANES)] * gi_v[pl.ds(o, _LANES)]
        for f in range(1, n_feat):
            p = f * b_per_w + o
            acc = acc + ws[f] * gu_v[pl.ds(p, _LANES)] * gi_v[pl.ds(p, _LANES)]
        out_v[pl.ds(o, _LANES)] = acc
        return carry

    lax.fori_loop(0, n_vec, compute_body, 0)

    pltpu.sync_copy(out_v, out_hbm.at[pl.ds(base, b_per_w)])


def kernel(user_ids, item_ids, user_table, item_table, predict_w):
    batch = user_ids.shape[0]
    n_rows, n_feat = user_table.shape
    info = plsc.get_sparse_core_info()
    n_workers = info.num_cores * info.num_subcores
    b_per_w = batch // n_workers

    uids = user_ids.astype(jnp.int32)
    iids = item_ids.astype(jnp.int32)
    w = predict_w.reshape(n_feat).astype(jnp.float32)
    ut = user_table.T
    it = item_table.T

    n_blocks = (n_rows + _UBLK - 1) // _UBLK
    out_rows_blk = (_UBLK // _QBLK) * n_feat
    flat_len = n_blocks * out_rows_blk * _QBLK

    detile = pl.pallas_call(
        functools.partial(_detile_body, n_feat),
        grid=(n_blocks,),
        in_specs=[
            pl.BlockSpec((n_feat, _UBLK), lambda g: (0, g)),
            pl.BlockSpec((n_feat, _UBLK), lambda g: (0, g)),
        ],
        out_specs=[
            pl.BlockSpec((out_rows_blk, _QBLK), lambda g: (g, 0)),
            pl.BlockSpec((out_rows_blk, _QBLK), lambda g: (g, 0)),
        ],
        out_shape=[
            jax.ShapeDtypeStruct((n_blocks * out_rows_blk, _QBLK), jnp.float32),
            jax.ShapeDtypeStruct((n_blocks * out_rows_blk, _QBLK), jnp.float32),
        ],
    )
    ut2, it2 = detile(ut, it)
    ut_flat = ut2.reshape(flat_len)
    it_flat = it2.reshape(flat_len)

    mesh = plsc.VectorSubcoreMesh(core_axis_name="c", subcore_axis_name="s")
    run = pl.kernel(
        functools.partial(_mf_kernel, b_per_w, n_feat),
        mesh=mesh,
        compiler_params=pltpu.CompilerParams(use_tc_tiling_on_sc=False),
        out_type=jax.ShapeDtypeStruct((batch,), jnp.float32),
        scratch_types=[
            pltpu.VMEM((2, b_per_w), jnp.int32),
            pltpu.VMEM((b_per_w * n_feat // _CHUNK, _CHUNK), jnp.int32),
            pltpu.VMEM((b_per_w * n_feat // _CHUNK, _CHUNK), jnp.int32),
            pltpu.VMEM((b_per_w * n_feat,), jnp.float32),
            pltpu.VMEM((b_per_w * n_feat,), jnp.float32),
            pltpu.VMEM((n_feat,), jnp.float32),
            pltpu.VMEM((b_per_w,), jnp.float32),
            pltpu.SemaphoreType.DMA,
        ],
    )
    return run(uids, iids, ut_flat, it_flat, w)


# speedup vs baseline: 1.0045x; 1.0045x over previous
import functools

import jax
import jax.numpy as jnp
from jax import lax
from jax.experimental import pallas as pl
from jax.experimental.pallas import tpu as pltpu
from jax.experimental.pallas import tpu_sc as plsc

_LANES = 16
_CHUNK = 128
_RING = 16
_UBLK = 32768
_QBLK = 128


def _detile_body(n_feat, ut_ref, it_ref, ou_ref, oi_ref):
    for q in range(_UBLK // _QBLK):
        dst = pl.ds(q * n_feat, n_feat)
        src = pl.ds(q * _QBLK, _QBLK)
        ou_ref[dst, :] = ut_ref[:, src]
        oi_ref[dst, :] = it_ref[:, src]


def _mf_kernel(b_per_w, n_feat, uids_hbm, iids_hbm, ut_hbm, it_hbm,
               w_hbm, out_hbm, idx_v, widx_u, widx_i, gu_v, gi_v, w_v, out_v,
               sem):
    wid = lax.axis_index("s") * 2 + lax.axis_index("c")
    base = wid * b_per_w

    pltpu.sync_copy(uids_hbm.at[pl.ds(base, b_per_w)], idx_v.at[0])
    pltpu.sync_copy(iids_hbm.at[pl.ds(base, b_per_w)], idx_v.at[1])
    pltpu.sync_copy(w_hbm, w_v)

    n_vec = b_per_w // _LANES
    n_chunks = (b_per_w * n_feat) // _CHUNK
    per_row = _CHUNK // _LANES

    def build_body(c, carry):
        uvec = idx_v[0, pl.ds(c * _LANES, _LANES)]
        ivec = idx_v[1, pl.ds(c * _LANES, _LANES)]
        ubase = ((uvec >> 7) << 12) + (uvec & 127)
        ibase = ((ivec >> 7) << 12) + (ivec & 127)
        for f in range(n_feat):
            pos = f * b_per_w + c * _LANES
            row = pos // _CHUNK
            off = (c % per_row) * _LANES
            widx_u[row, pl.ds(off, _LANES)] = ubase + f * _QBLK
            widx_i[row, pl.ds(off, _LANES)] = ibase + f * _QBLK
        return carry

    lax.fori_loop(0, n_vec, build_body, 0)

    def fire(j):
        pltpu.make_async_copy(
            ut_hbm.at[widx_u.at[j]],
            gu_v.at[pl.ds(j * _CHUNK, _CHUNK)], sem).start()
        pltpu.make_async_copy(
            it_hbm.at[widx_i.at[j]],
            gi_v.at[pl.ds(j * _CHUNK, _CHUNK)], sem).start()

    def drain_pair():
        pltpu.make_async_copy(
            out_hbm.at[pl.ds(0, _CHUNK)],
            gu_v.at[pl.ds(0, _CHUNK)], sem).wait()
        pltpu.make_async_copy(
            out_hbm.at[pl.ds(0, _CHUNK)],
            gi_v.at[pl.ds(0, _CHUNK)], sem).wait()

    def pipe_body(j, carry):
        fire(j)

        @pl.when(j >= _RING)
        def _():
            drain_pair()

        return carry

    lax.fori_loop(0, n_chunks, pipe_body, 0)
    for _ in range(_RING):
        drain_pair()

    lane = lax.iota(jnp.int32, _LANES)
    dnums = lax.GatherDimensionNumbers(
        offset_dims=(), collapsed_slice_dims=(0,), start_index_map=(0,))

    def permute(t, p):
        return lax.gather(t, p[:, None], dnums, slice_sizes=(1,),
                          mode=lax.GatherScatterMode.PROMISE_IN_BOUNDS)

    wregs = [w_v[pl.ds(k * _LANES, _LANES)] for k in range(n_feat // _LANES)]
    ws = [permute(wregs[f // _LANES], lane * 0 + (f % _LANES))
          for f in range(n_feat)]

    def compute_body(c, carry):
        o = c * _LANES
        acc = ws[0] * gu_v[pl.ds(o, _LANES)] * gi_v[pl.ds(o, _LANES)]
        for f in range(1, n_feat):
            p = f * b_per_w + o
            acc = acc + ws[f] * gu_v[pl.ds(p, _LANES)] * gi_v[pl.ds(p, _LANES)]
        out_v[pl.ds(o, _LANES)] = acc
        return carry

    lax.fori_loop(0, n_vec, compute_body, 0)

    pltpu.sync_copy(out_v, out_hbm.at[pl.ds(base, b_per_w)])


def kernel(user_ids, item_ids, user_table, item_table, predict_w):
    batch = user_ids.shape[0]
    n_rows, n_feat = user_table.shape
    info = plsc.get_sparse_core_info()
    n_workers = info.num_cores * info.num_subcores
    b_per_w = batch // n_workers

    uids = user_ids.astype(jnp.int32)
    iids = item_ids.astype(jnp.int32)
    w = predict_w.reshape(n_feat).astype(jnp.float32)
    ut = user_table.T
    it = item_table.T

    n_blocks = (n_rows + _UBLK - 1) // _UBLK
    out_rows_blk = (_UBLK // _QBLK) * n_feat
    flat_len = n_blocks * out_rows_blk * _QBLK

    detile = pl.pallas_call(
        functools.partial(_detile_body, n_feat),
        grid=(n_blocks,),
        in_specs=[
            pl.BlockSpec((n_feat, _UBLK), lambda g: (0, g)),
            pl.BlockSpec((n_feat, _UBLK), lambda g: (0, g)),
        ],
        out_specs=[
            pl.BlockSpec((out_rows_blk, _QBLK), lambda g: (g, 0)),
            pl.BlockSpec((out_rows_blk, _QBLK), lambda g: (g, 0)),
        ],
        out_shape=[
            jax.ShapeDtypeStruct((n_blocks * out_rows_blk, _QBLK), jnp.float32),
            jax.ShapeDtypeStruct((n_blocks * out_rows_blk, _QBLK), jnp.float32),
        ],
    )
    ut2, it2 = detile(ut, it)
    ut_flat = ut2.reshape(flat_len)
    it_flat = it2.reshape(flat_len)

    mesh = plsc.VectorSubcoreMesh(core_axis_name="c", subcore_axis_name="s")
    run = pl.kernel(
        functools.partial(_mf_kernel, b_per_w, n_feat),
        mesh=mesh,
        compiler_params=pltpu.CompilerParams(use_tc_tiling_on_sc=False),
        out_type=jax.ShapeDtypeStruct((batch,), jnp.float32),
        scratch_types=[
            pltpu.VMEM((2, b_per_w), jnp.int32),
            pltpu.VMEM((b_per_w * n_feat // _CHUNK, _CHUNK), jnp.int32),
            pltpu.VMEM((b_per_w * n_feat // _CHUNK, _CHUNK), jnp.int32),
            pltpu.VMEM((b_per_w * n_feat,), jnp.float32),
            pltpu.VMEM((b_per_w * n_feat,), jnp.float32),
            pltpu.VMEM((n_feat,), jnp.float32),
            pltpu.VMEM((b_per_w,), jnp.float32),
            pltpu.SemaphoreType.DMA,
        ],
    )
    return run(uids, iids, ut_flat, it_flat, w)
